# initial kernel scaffold (unmeasured)
import jax
import jax.numpy as jnp
from jax import lax
from jax.experimental import pallas as pl
from jax.experimental.pallas import tpu as pltpu

N_DEV = 4
SQ = 256
SKV = 4096
HQ_LOCAL = 8
DH = 128
D_MODEL = 1024
QBLK = 64
NGROUP = 4
KV_PER_GROUP = SKV // NGROUP
NKBLK = KV_PER_GROUP // QBLK
SCALE = 0.08838834764831843


def kernel(x, Wq, K_ext, V_ext, Wo):
    def body(x_ref, wq_ref, k_ref, v_ref, wo_ref, out_ref,
             wq_v, wo_v, kg, vg, q_s, ctx_s, comm,
             sem_w, sem_o, sem_kv, send_sems, recv_sems):
        my = lax.axis_index("i")
        right = lax.rem(my + 1, N_DEV)

        wq_dma = pltpu.make_async_copy(
            wq_ref.at[:, pl.ds(my * D_MODEL, D_MODEL)], wq_v, sem_w)
        wq_dma.start()
        wo_dma = pltpu.make_async_copy(
            wo_ref.at[pl.ds(my * D_MODEL, D_MODEL), :], wo_v, sem_o)
        wo_dma.start()

        kv_dmas = []
        for g in range(NGROUP):
            for t in range(NKBLK):
                b = g + NGROUP * t
                kv_dmas.append(pltpu.make_async_copy(
                    k_ref.at[0, pl.ds(b * QBLK, QBLK)],
                    kg.at[g, pl.ds(t * QBLK, QBLK)], sem_kv))
                kv_dmas.append(pltpu.make_async_copy(
                    v_ref.at[0, pl.ds(b * QBLK, QBLK)],
                    vg.at[g, pl.ds(t * QBLK, QBLK)], sem_kv))
        for d in kv_dmas:
            d.start()

        wq_dma.wait()
        q_s[...] = jnp.dot(
            x_ref[0], wq_v[...], preferred_element_type=jnp.float32) * SCALE

        for d in kv_dmas:
            d.wait()
        for g in range(NGROUP):
            for h in range(HQ_LOCAL):
                qgh = q_s[g * QBLK:(g + 1) * QBLK, h * DH:(h + 1) * DH]
                kgh = kg[g, :, h, :]
                s = lax.dot_general(
                    qgh, kgh, (((1,), (1,)), ((), ())),
                    preferred_element_type=jnp.float32)
                m = jnp.max(s, axis=1, keepdims=True)
                e = jnp.exp(s - m)
                w = e / jnp.sum(e, axis=1, keepdims=True)
                vgh = vg[g, :, h, :]
                ctx_s[g * QBLK:(g + 1) * QBLK, h * DH:(h + 1) * DH] = (
                    lax.dot_general(
                        w, vgh, (((1,), (0,)), ((), ())),
                        preferred_element_type=jnp.float32))

        wo_dma.wait()
        out_ref[0] = jnp.dot(
            ctx_s[...], wo_v[...], preferred_element_type=jnp.float32)

        for hop in range(N_DEV - 1):
            src = out_ref.at[0] if hop == 0 else comm.at[hop - 1]
            rdma = pltpu.make_async_remote_copy(
                src_ref=src,
                dst_ref=comm.at[hop],
                send_sem=send_sems.at[hop],
                recv_sem=recv_sems.at[hop],
                device_id=(right,),
                device_id_type=pl.DeviceIdType.MESH,
            )
            rdma.start()
            rdma.wait()
            out_ref[0] = out_ref[0] + comm[hop]

    out_shape = jax.ShapeDtypeStruct((1, SQ, D_MODEL), jnp.float32)
    return pl.pallas_call(
        body,
        out_shape=out_shape,
        in_specs=[
            pl.BlockSpec(memory_space=pltpu.VMEM),
            pl.BlockSpec(memory_space=pltpu.ANY),
            pl.BlockSpec(memory_space=pltpu.ANY),
            pl.BlockSpec(memory_space=pltpu.ANY),
            pl.BlockSpec(memory_space=pltpu.ANY),
        ],
        out_specs=pl.BlockSpec(memory_space=pltpu.VMEM),
        scratch_shapes=[
            pltpu.VMEM((D_MODEL, D_MODEL), jnp.float32),
            pltpu.VMEM((D_MODEL, D_MODEL), jnp.float32),
            pltpu.VMEM((NGROUP, KV_PER_GROUP, HQ_LOCAL, DH), jnp.float32),
            pltpu.VMEM((NGROUP, KV_PER_GROUP, HQ_LOCAL, DH), jnp.float32),
            pltpu.VMEM((SQ, D_MODEL), jnp.float32),
            pltpu.VMEM((SQ, D_MODEL), jnp.float32),
            pltpu.VMEM((N_DEV - 1, SQ, D_MODEL), jnp.float32),
            pltpu.SemaphoreType.DMA,
            pltpu.SemaphoreType.DMA,
            pltpu.SemaphoreType.DMA,
            pltpu.SemaphoreType.DMA((N_DEV - 1,)),
            pltpu.SemaphoreType.DMA((N_DEV - 1,)),
        ],
        compiler_params=pltpu.CompilerParams(collective_id=0),
    )(x, Wq, K_ext, V_ext, Wo)


# baseline (device time: 87915 ns/iter reference)
import jax
import jax.numpy as jnp
from jax import lax
from jax.experimental import pallas as pl
from jax.experimental.pallas import tpu as pltpu

N_DEV = 4
SQ = 256
SKV = 4096
HQ_LOCAL = 8
DH = 128
D_MODEL = 1024
QBLK = 64
NGROUP = 4
KV_PER_GROUP = SKV // NGROUP
NKBLK = KV_PER_GROUP // QBLK
SCALE = 0.08838834764831843


def kernel(x, Wq, K_ext, V_ext, Wo):
    def body(x_ref, wq_ref, k_ref, v_ref, wo_ref, out_ref,
             wq_v, wo_v, kg, vg, q_s, ctx_s, comm,
             sem_w, sem_o, sem_kv, send_sems, recv_sems):
        my = lax.axis_index("i")
        right = lax.rem(my + 1, N_DEV)

        wq_dma = pltpu.make_async_copy(
            wq_ref.at[:, pl.ds(my * D_MODEL, D_MODEL)], wq_v, sem_w)
        wq_dma.start()
        wo_dma = pltpu.make_async_copy(
            wo_ref.at[pl.ds(my * D_MODEL, D_MODEL), :], wo_v, sem_o)
        wo_dma.start()

        kv_dmas = []
        for g in range(NGROUP):
            for t in range(NKBLK):
                b = g + NGROUP * t
                kv_dmas.append(pltpu.make_async_copy(
                    k_ref.at[0, pl.ds(b * QBLK, QBLK)],
                    kg.at[g, pl.ds(t * QBLK, QBLK)], sem_kv))
                kv_dmas.append(pltpu.make_async_copy(
                    v_ref.at[0, pl.ds(b * QBLK, QBLK)],
                    vg.at[g, pl.ds(t * QBLK, QBLK)], sem_kv))
        for d in kv_dmas:
            d.start()

        wq_dma.wait()
        q_s[...] = jnp.dot(
            x_ref[0], wq_v[...], preferred_element_type=jnp.float32) * SCALE

        for d in kv_dmas:
            d.wait()
        for g in range(NGROUP):
            for h in range(HQ_LOCAL):
                qgh = q_s[g * QBLK:(g + 1) * QBLK, h * DH:(h + 1) * DH]
                kgh = kg[g, :, h, :]
                s = lax.dot_general(
                    qgh, kgh, (((1,), (1,)), ((), ())),
                    preferred_element_type=jnp.float32)
                m = jnp.max(s, axis=1, keepdims=True)
                e = jnp.exp(s - m)
                w = e / jnp.sum(e, axis=1, keepdims=True)
                vgh = vg[g, :, h, :]
                ctx_s[g * QBLK:(g + 1) * QBLK, h * DH:(h + 1) * DH] = (
                    lax.dot_general(
                        w, vgh, (((1,), (0,)), ((), ())),
                        preferred_element_type=jnp.float32))

        wo_dma.wait()
        out_ref[0] = jnp.dot(
            ctx_s[...], wo_v[...], preferred_element_type=jnp.float32)

        for hop in range(N_DEV - 1):
            src = out_ref.at[0] if hop == 0 else comm.at[hop - 1]
            rdma = pltpu.make_async_remote_copy(
                src_ref=src,
                dst_ref=comm.at[hop],
                send_sem=send_sems.at[hop],
                recv_sem=recv_sems.at[hop],
                device_id=(right,),
                device_id_type=pl.DeviceIdType.MESH,
            )
            rdma.start()
            rdma.wait()
            out_ref[0] = out_ref[0] + comm[hop]

    out_shape = jax.ShapeDtypeStruct((1, SQ, D_MODEL), jnp.float32)
    return pl.pallas_call(
        body,
        out_shape=out_shape,
        in_specs=[
            pl.BlockSpec(memory_space=pltpu.VMEM),
            pl.BlockSpec(memory_space=pl.ANY),
            pl.BlockSpec(memory_space=pl.ANY),
            pl.BlockSpec(memory_space=pl.ANY),
            pl.BlockSpec(memory_space=pl.ANY),
        ],
        out_specs=pl.BlockSpec(memory_space=pltpu.VMEM),
        scratch_shapes=[
            pltpu.VMEM((D_MODEL, D_MODEL), jnp.float32),
            pltpu.VMEM((D_MODEL, D_MODEL), jnp.float32),
            pltpu.VMEM((NGROUP, KV_PER_GROUP, HQ_LOCAL, DH), jnp.float32),
            pltpu.VMEM((NGROUP, KV_PER_GROUP, HQ_LOCAL, DH), jnp.float32),
            pltpu.VMEM((SQ, D_MODEL), jnp.float32),
            pltpu.VMEM((SQ, D_MODEL), jnp.float32),
            pltpu.VMEM((N_DEV - 1, SQ, D_MODEL), jnp.float32),
            pltpu.SemaphoreType.DMA,
            pltpu.SemaphoreType.DMA,
            pltpu.SemaphoreType.DMA,
            pltpu.SemaphoreType.DMA((N_DEV - 1,)),
            pltpu.SemaphoreType.DMA((N_DEV - 1,)),
        ],
        compiler_params=pltpu.CompilerParams(
            vmem_limit_bytes=100 * 1024 * 1024,
        ),
    )(x, Wq, K_ext, V_ext, Wo)


# device time: 72298 ns/iter; 1.2160x vs baseline; 1.2160x over previous
import jax
import jax.numpy as jnp
from jax import lax
from jax.experimental import pallas as pl
from jax.experimental.pallas import tpu as pltpu

N_DEV = 4
SQ = 256
SKV = 4096
HQ_LOCAL = 8
DH = 128
D_MODEL = 1024
QBLK = 64
NGROUP = 4
KV_PER_GROUP = SKV // NGROUP
NKBLK = KV_PER_GROUP // QBLK
SCALE = 0.08838834764831843


def kernel(x, Wq, K_ext, V_ext, Wo):
    def body(x_ref, wq_ref, k_ref, v_ref, wo_ref, out_ref,
             wq_v, wo_v, kg, vg, q_s, ctx_s, rbuf1, rbuf2,
             sem_w, sem_o, sem_kv, send_sems, recv_sems):
        my = lax.axis_index("i")

        wq_dma = pltpu.make_async_copy(
            wq_ref.at[:, pl.ds(my * D_MODEL, D_MODEL)], wq_v, sem_w)
        wq_dma.start()
        wo_dma = pltpu.make_async_copy(
            wo_ref.at[pl.ds(my * D_MODEL, D_MODEL), :], wo_v, sem_o)
        wo_dma.start()

        kv_dmas = []
        for g in range(NGROUP):
            for t in range(NKBLK):
                b = g + NGROUP * t
                kv_dmas.append(pltpu.make_async_copy(
                    k_ref.at[0, pl.ds(b * QBLK, QBLK)],
                    kg.at[g, pl.ds(t * QBLK, QBLK)], sem_kv))
                kv_dmas.append(pltpu.make_async_copy(
                    v_ref.at[0, pl.ds(b * QBLK, QBLK)],
                    vg.at[g, pl.ds(t * QBLK, QBLK)], sem_kv))
        for d in kv_dmas:
            d.start()

        wq_dma.wait()
        q_s[...] = jnp.dot(
            x_ref[0], wq_v[...], preferred_element_type=jnp.float32) * SCALE

        for d in kv_dmas:
            d.wait()
        for g in range(NGROUP):
            for h in range(HQ_LOCAL):
                qgh = q_s[g * QBLK:(g + 1) * QBLK, h * DH:(h + 1) * DH]
                kgh = kg[g, :, h, :]
                s = lax.dot_general(
                    qgh, kgh, (((1,), (1,)), ((), ())),
                    preferred_element_type=jnp.float32)
                m = jnp.max(s, axis=1, keepdims=True)
                e = jnp.exp(s - m)
                w = e / jnp.sum(e, axis=1, keepdims=True)
                vgh = vg[g, :, h, :]
                ctx_s[g * QBLK:(g + 1) * QBLK, h * DH:(h + 1) * DH] = (
                    lax.dot_general(
                        w, vgh, (((1,), (0,)), ((), ())),
                        preferred_element_type=jnp.float32))

        wo_dma.wait()
        out_ref[0] = jnp.dot(
            ctx_s[...], wo_v[...], preferred_element_type=jnp.float32)

        pa = my ^ 1
        pb = 3 - my
        ho = jnp.where((my == 0) | (my == 3), 0, 1)
        qo = jnp.where(my < 2, 0, 1)
        keep_h = ho * 128
        send_h = (1 - ho) * 128
        keep_q = keep_h + qo * 64
        send_q = keep_h + (1 - qo) * 64

        r1 = pltpu.make_async_remote_copy(
            src_ref=out_ref.at[0, pl.ds(send_h, 128)],
            dst_ref=rbuf1,
            send_sem=send_sems.at[0], recv_sem=recv_sems.at[0],
            device_id=(pa,), device_id_type=pl.DeviceIdType.MESH)
        r1.start()
        r1.wait()
        out_ref[0, pl.ds(keep_h, 128)] = (
            out_ref[0, pl.ds(keep_h, 128)] + rbuf1[...])

        r2 = pltpu.make_async_remote_copy(
            src_ref=out_ref.at[0, pl.ds(send_q, 64)],
            dst_ref=rbuf2,
            send_sem=send_sems.at[1], recv_sem=recv_sems.at[1],
            device_id=(pb,), device_id_type=pl.DeviceIdType.MESH)
        r2.start()
        r2.wait()
        out_ref[0, pl.ds(keep_q, 64)] = (
            out_ref[0, pl.ds(keep_q, 64)] + rbuf2[...])

        r3 = pltpu.make_async_remote_copy(
            src_ref=out_ref.at[0, pl.ds(keep_q, 64)],
            dst_ref=out_ref.at[0, pl.ds(keep_q, 64)],
            send_sem=send_sems.at[2], recv_sem=recv_sems.at[2],
            device_id=(pb,), device_id_type=pl.DeviceIdType.MESH)
        r3.start()
        r3.wait()

        r4 = pltpu.make_async_remote_copy(
            src_ref=out_ref.at[0, pl.ds(keep_h, 128)],
            dst_ref=out_ref.at[0, pl.ds(keep_h, 128)],
            send_sem=send_sems.at[3], recv_sem=recv_sems.at[3],
            device_id=(pa,), device_id_type=pl.DeviceIdType.MESH)
        r4.start()
        r4.wait()

    out_shape = jax.ShapeDtypeStruct((1, SQ, D_MODEL), jnp.float32)
    return pl.pallas_call(
        body,
        out_shape=out_shape,
        in_specs=[
            pl.BlockSpec(memory_space=pltpu.VMEM),
            pl.BlockSpec(memory_space=pl.ANY),
            pl.BlockSpec(memory_space=pl.ANY),
            pl.BlockSpec(memory_space=pl.ANY),
            pl.BlockSpec(memory_space=pl.ANY),
        ],
        out_specs=pl.BlockSpec(memory_space=pltpu.VMEM),
        scratch_shapes=[
            pltpu.VMEM((D_MODEL, D_MODEL), jnp.float32),
            pltpu.VMEM((D_MODEL, D_MODEL), jnp.float32),
            pltpu.VMEM((NGROUP, KV_PER_GROUP, HQ_LOCAL, DH), jnp.float32),
            pltpu.VMEM((NGROUP, KV_PER_GROUP, HQ_LOCAL, DH), jnp.float32),
            pltpu.VMEM((SQ, D_MODEL), jnp.float32),
            pltpu.VMEM((SQ, D_MODEL), jnp.float32),
            pltpu.VMEM((SQ // 2, D_MODEL), jnp.float32),
            pltpu.VMEM((SQ // 4, D_MODEL), jnp.float32),
            pltpu.SemaphoreType.DMA,
            pltpu.SemaphoreType.DMA,
            pltpu.SemaphoreType.DMA,
            pltpu.SemaphoreType.DMA((4,)),
            pltpu.SemaphoreType.DMA((4,)),
        ],
        compiler_params=pltpu.CompilerParams(
            vmem_limit_bytes=100 * 1024 * 1024,
        ),
    )(x, Wq, K_ext, V_ext, Wo)


# device time: 64399 ns/iter; 1.3652x vs baseline; 1.1227x over previous
import jax
import jax.numpy as jnp
from jax import lax
from jax.experimental import pallas as pl
from jax.experimental.pallas import tpu as pltpu

N_DEV = 4
SQ = 256
SKV = 4096
HQ_LOCAL = 8
DH = 128
D_MODEL = 1024
QBLK = 64
NGROUP = 4
KV_PER_GROUP = SKV // NGROUP
NKBLK = KV_PER_GROUP // QBLK
SCALE = 0.08838834764831843


def kernel(x, Wq, K_ext, V_ext, Wo):
    def body(x_ref, wq_ref, k_ref, v_ref, wo_ref, out_ref,
             wq_v, wo_v, kg, vg, q_s, ctx_s,
             sbuf1, sbuf2, rbuf1, rbuf2, rbuf3, rbuf4,
             sem_w, sem_o, sem_kv, send_sems, recv_sems):
        my = lax.axis_index("i")

        wq_dma = pltpu.make_async_copy(
            wq_ref.at[:, pl.ds(my * D_MODEL, D_MODEL)], wq_v, sem_w)
        wq_dma.start()
        wo_dma = pltpu.make_async_copy(
            wo_ref.at[pl.ds(my * D_MODEL, D_MODEL), :], wo_v, sem_o)
        wo_dma.start()

        kv_dmas = [[] for _ in range(NGROUP)]
        for g in range(NGROUP):
            for t in range(NKBLK):
                b = g + NGROUP * t
                kv_dmas[g].append(pltpu.make_async_copy(
                    k_ref.at[0, pl.ds(b * QBLK, QBLK)],
                    kg.at[g, pl.ds(t * QBLK, QBLK)], sem_kv.at[g]))
                kv_dmas[g].append(pltpu.make_async_copy(
                    v_ref.at[0, pl.ds(b * QBLK, QBLK)],
                    vg.at[g, pl.ds(t * QBLK, QBLK)], sem_kv.at[g]))
        for dg in kv_dmas:
            for d in dg:
                d.start()

        wq_dma.wait()
        q_s[...] = jnp.dot(
            x_ref[0], wq_v[...], preferred_element_type=jnp.float32) * SCALE

        for g in range(NGROUP):
            for d in kv_dmas[g]:
                d.wait()
            for h in range(HQ_LOCAL):
                qgh = q_s[g * QBLK:(g + 1) * QBLK, h * DH:(h + 1) * DH]
                kgh = kg[g, :, h, :]
                s = lax.dot_general(
                    qgh, kgh, (((1,), (1,)), ((), ())),
                    preferred_element_type=jnp.float32)
                m = jnp.max(s, axis=1, keepdims=True)
                e = jnp.exp(s - m)
                w = e / jnp.sum(e, axis=1, keepdims=True)
                vgh = vg[g, :, h, :]
                ctx_s[g * QBLK:(g + 1) * QBLK, h * DH:(h + 1) * DH] = (
                    lax.dot_general(
                        w, vgh, (((1,), (0,)), ((), ())),
                        preferred_element_type=jnp.float32))

        wo_dma.wait()
        out_ref[0] = jnp.dot(
            ctx_s[...], wo_v[...], preferred_element_type=jnp.float32)

        pa = my ^ 1
        pb = 3 - my
        ho = jnp.where((my == 0) | (my == 3), 0, 1)
        qo = jnp.where(my < 2, 0, 1)
        keep_h = ho * 128
        send_h = (1 - ho) * 128
        keep_q = keep_h + qo * 64
        send_q = keep_h + (1 - qo) * 64


        sbuf1[...] = out_ref[0, pl.ds(send_h, 128)].astype(jnp.bfloat16)
        r1 = pltpu.make_async_remote_copy(
            src_ref=sbuf1,
            dst_ref=rbuf1,
            send_sem=send_sems.at[0], recv_sem=recv_sems.at[0],
            device_id=(pa,), device_id_type=pl.DeviceIdType.MESH)
        r1.start()
        r1.wait()
        out_ref[0, pl.ds(keep_h, 128)] = (
            out_ref[0, pl.ds(keep_h, 128)] + rbuf1[...].astype(jnp.float32))

        sbuf2[...] = out_ref[0, pl.ds(send_q, 64)].astype(jnp.bfloat16)
        r2 = pltpu.make_async_remote_copy(
            src_ref=sbuf2,
            dst_ref=rbuf2,
            send_sem=send_sems.at[1], recv_sem=recv_sems.at[1],
            device_id=(pb,), device_id_type=pl.DeviceIdType.MESH)
        r2.start()
        r2.wait()
        out_ref[0, pl.ds(keep_q, 64)] = (
            out_ref[0, pl.ds(keep_q, 64)] + rbuf2[...].astype(jnp.float32))

        sbuf2[...] = out_ref[0, pl.ds(keep_q, 64)].astype(jnp.bfloat16)
        r3 = pltpu.make_async_remote_copy(
            src_ref=sbuf2,
            dst_ref=rbuf3,
            send_sem=send_sems.at[2], recv_sem=recv_sems.at[2],
            device_id=(pb,), device_id_type=pl.DeviceIdType.MESH)
        r3.start()
        r3.wait()
        out_ref[0, pl.ds(send_q, 64)] = rbuf3[...].astype(jnp.float32)

        sbuf1[...] = out_ref[0, pl.ds(keep_h, 128)].astype(jnp.bfloat16)
        r4 = pltpu.make_async_remote_copy(
            src_ref=sbuf1,
            dst_ref=rbuf4,
            send_sem=send_sems.at[3], recv_sem=recv_sems.at[3],
            device_id=(pa,), device_id_type=pl.DeviceIdType.MESH)
        r4.start()
        r4.wait()
        out_ref[0, pl.ds(send_h, 128)] = rbuf4[...].astype(jnp.float32)

    out_shape = jax.ShapeDtypeStruct((1, SQ, D_MODEL), jnp.float32)
    return pl.pallas_call(
        body,
        out_shape=out_shape,
        in_specs=[
            pl.BlockSpec(memory_space=pltpu.VMEM),
            pl.BlockSpec(memory_space=pl.ANY),
            pl.BlockSpec(memory_space=pl.ANY),
            pl.BlockSpec(memory_space=pl.ANY),
            pl.BlockSpec(memory_space=pl.ANY),
        ],
        out_specs=pl.BlockSpec(memory_space=pltpu.VMEM),
        scratch_shapes=[
            pltpu.VMEM((D_MODEL, D_MODEL), jnp.float32),
            pltpu.VMEM((D_MODEL, D_MODEL), jnp.float32),
            pltpu.VMEM((NGROUP, KV_PER_GROUP, HQ_LOCAL, DH), jnp.float32),
            pltpu.VMEM((NGROUP, KV_PER_GROUP, HQ_LOCAL, DH), jnp.float32),
            pltpu.VMEM((SQ, D_MODEL), jnp.float32),
            pltpu.VMEM((SQ, D_MODEL), jnp.float32),
            pltpu.VMEM((SQ // 2, D_MODEL), jnp.bfloat16),
            pltpu.VMEM((SQ // 4, D_MODEL), jnp.bfloat16),
            pltpu.VMEM((SQ // 2, D_MODEL), jnp.bfloat16),
            pltpu.VMEM((SQ // 4, D_MODEL), jnp.bfloat16),
            pltpu.VMEM((SQ // 4, D_MODEL), jnp.bfloat16),
            pltpu.VMEM((SQ // 2, D_MODEL), jnp.bfloat16),
            pltpu.SemaphoreType.DMA,
            pltpu.SemaphoreType.DMA,
            pltpu.SemaphoreType.DMA((NGROUP,)),
            pltpu.SemaphoreType.DMA((4,)),
            pltpu.SemaphoreType.DMA((4,)),
        ],
        compiler_params=pltpu.CompilerParams(
            vmem_limit_bytes=100 * 1024 * 1024,
        ),
    )(x, Wq, K_ext, V_ext, Wo)


# device time: 59783 ns/iter; 1.4706x vs baseline; 1.0772x over previous
import jax
import jax.numpy as jnp
from jax import lax
from jax.experimental import pallas as pl
from jax.experimental.pallas import tpu as pltpu

N_DEV = 4
SQ = 256
SKV = 4096
HQ_LOCAL = 8
DH = 128
D_MODEL = 1024
QBLK = 64
NGROUP = 4
KV_PER_GROUP = SKV // NGROUP
NKBLK = KV_PER_GROUP // QBLK
SCALE = 0.08838834764831843


def kernel(x, Wq, K_ext, V_ext, Wo):
    def body(x_ref, wq_ref, k_ref, v_ref, wo_ref, out_ref,
             wq_v, wo_v, kg, vg, q_s, ctx_s,
             sbuf1, sbuf2, rbuf1, rbuf2, rbuf3, rbuf4,
             sem_w, sem_o, sem_kv, send_sems, recv_sems):
        my = lax.axis_index("i")

        wq_dma = pltpu.make_async_copy(
            wq_ref.at[:, pl.ds(my * D_MODEL, D_MODEL)], wq_v, sem_w)
        wq_dma.start()
        wo_dma = pltpu.make_async_copy(
            wo_ref.at[pl.ds(my * D_MODEL, D_MODEL), :], wo_v, sem_o)
        wo_dma.start()

        pa = my ^ 1
        pb = 3 - my
        ho = jnp.where((my == 0) | (my == 3), 0, 1)
        qo = jnp.where(my < 2, 0, 1)
        keep_h = ho * 128
        send_h = (1 - ho) * 128
        keep_q = keep_h + qo * 64
        send_q = keep_h + (1 - qo) * 64

        g0 = 2 * (1 - ho)
        g1 = 2 * ho
        order = [g0, g0 + 1, g1, g1 + 1]

        kv_dmas = [[] for _ in range(NGROUP)]
        for s in range(NGROUP):
            g = order[s]
            for t in range(NKBLK):
                boff = (g + NGROUP * t) * QBLK
                kv_dmas[s].append(pltpu.make_async_copy(
                    k_ref.at[0, pl.ds(boff, QBLK)],
                    kg.at[s, pl.ds(t * QBLK, QBLK)], sem_kv.at[s]))
                kv_dmas[s].append(pltpu.make_async_copy(
                    v_ref.at[0, pl.ds(boff, QBLK)],
                    vg.at[s, pl.ds(t * QBLK, QBLK)], sem_kv.at[s]))
        for dg in kv_dmas:
            for d in dg:
                d.start()

        wq_dma.wait()
        q_s[...] = jnp.dot(
            x_ref[0], wq_v[...], preferred_element_type=jnp.float32) * SCALE

        def attn_slot(s):
            g = order[s]
            for d in kv_dmas[s]:
                d.wait()
            for h in range(HQ_LOCAL):
                qgh = q_s[pl.ds(g * QBLK, QBLK), h * DH:(h + 1) * DH]
                kgh = kg[s, :, h, :]
                sc = lax.dot_general(
                    qgh, kgh, (((1,), (1,)), ((), ())),
                    preferred_element_type=jnp.float32)
                m = jnp.max(sc, axis=1, keepdims=True)
                e = jnp.exp(sc - m)
                w = e / jnp.sum(e, axis=1, keepdims=True)
                vgh = vg[s, :, h, :]
                ctx_s[pl.ds(g * QBLK, QBLK), h * DH:(h + 1) * DH] = (
                    lax.dot_general(
                        w, vgh, (((1,), (0,)), ((), ())),
                        preferred_element_type=jnp.float32))

        attn_slot(0)
        attn_slot(1)
        wo_dma.wait()
        out_ref[0, pl.ds(send_h, 128)] = jnp.dot(
            ctx_s[pl.ds(send_h, 128)], wo_v[...],
            preferred_element_type=jnp.float32)

        sbuf1[...] = out_ref[0, pl.ds(send_h, 128)].astype(jnp.bfloat16)
        r1 = pltpu.make_async_remote_copy(
            src_ref=sbuf1,
            dst_ref=rbuf1,
            send_sem=send_sems.at[0], recv_sem=recv_sems.at[0],
            device_id=(pa,), device_id_type=pl.DeviceIdType.MESH)
        r1.start()

        attn_slot(2)
        attn_slot(3)
        out_ref[0, pl.ds(keep_h, 128)] = jnp.dot(
            ctx_s[pl.ds(keep_h, 128)], wo_v[...],
            preferred_element_type=jnp.float32)

        r1.wait()
        out_ref[0, pl.ds(keep_h, 128)] = (
            out_ref[0, pl.ds(keep_h, 128)] + rbuf1[...].astype(jnp.float32))

        sbuf2[...] = out_ref[0, pl.ds(send_q, 64)].astype(jnp.bfloat16)
        r2 = pltpu.make_async_remote_copy(
            src_ref=sbuf2,
            dst_ref=rbuf2,
            send_sem=send_sems.at[1], recv_sem=recv_sems.at[1],
            device_id=(pb,), device_id_type=pl.DeviceIdType.MESH)
        r2.start()
        r2.wait()
        out_ref[0, pl.ds(keep_q, 64)] = (
            out_ref[0, pl.ds(keep_q, 64)] + rbuf2[...].astype(jnp.float32))

        sbuf2[...] = out_ref[0, pl.ds(keep_q, 64)].astype(jnp.bfloat16)
        r3 = pltpu.make_async_remote_copy(
            src_ref=sbuf2,
            dst_ref=rbuf3,
            send_sem=send_sems.at[2], recv_sem=recv_sems.at[2],
            device_id=(pb,), device_id_type=pl.DeviceIdType.MESH)
        r3.start()
        r3.wait()
        out_ref[0, pl.ds(send_q, 64)] = rbuf3[...].astype(jnp.float32)

        sbuf1[...] = out_ref[0, pl.ds(keep_h, 128)].astype(jnp.bfloat16)
        r4 = pltpu.make_async_remote_copy(
            src_ref=sbuf1,
            dst_ref=rbuf4,
            send_sem=send_sems.at[3], recv_sem=recv_sems.at[3],
            device_id=(pa,), device_id_type=pl.DeviceIdType.MESH)
        r4.start()
        r4.wait()
        out_ref[0, pl.ds(send_h, 128)] = rbuf4[...].astype(jnp.float32)

    out_shape = jax.ShapeDtypeStruct((1, SQ, D_MODEL), jnp.float32)
    return pl.pallas_call(
        body,
        out_shape=out_shape,
        in_specs=[
            pl.BlockSpec(memory_space=pltpu.VMEM),
            pl.BlockSpec(memory_space=pl.ANY),
            pl.BlockSpec(memory_space=pl.ANY),
            pl.BlockSpec(memory_space=pl.ANY),
            pl.BlockSpec(memory_space=pl.ANY),
        ],
        out_specs=pl.BlockSpec(memory_space=pltpu.VMEM),
        scratch_shapes=[
            pltpu.VMEM((D_MODEL, D_MODEL), jnp.float32),
            pltpu.VMEM((D_MODEL, D_MODEL), jnp.float32),
            pltpu.VMEM((NGROUP, KV_PER_GROUP, HQ_LOCAL, DH), jnp.float32),
            pltpu.VMEM((NGROUP, KV_PER_GROUP, HQ_LOCAL, DH), jnp.float32),
            pltpu.VMEM((SQ, D_MODEL), jnp.float32),
            pltpu.VMEM((SQ, D_MODEL), jnp.float32),
            pltpu.VMEM((SQ // 2, D_MODEL), jnp.bfloat16),
            pltpu.VMEM((SQ // 4, D_MODEL), jnp.bfloat16),
            pltpu.VMEM((SQ // 2, D_MODEL), jnp.bfloat16),
            pltpu.VMEM((SQ // 4, D_MODEL), jnp.bfloat16),
            pltpu.VMEM((SQ // 4, D_MODEL), jnp.bfloat16),
            pltpu.VMEM((SQ // 2, D_MODEL), jnp.bfloat16),
            pltpu.SemaphoreType.DMA,
            pltpu.SemaphoreType.DMA,
            pltpu.SemaphoreType.DMA((NGROUP,)),
            pltpu.SemaphoreType.DMA((4,)),
            pltpu.SemaphoreType.DMA((4,)),
        ],
        compiler_params=pltpu.CompilerParams(
            vmem_limit_bytes=100 * 1024 * 1024,
        ),
    )(x, Wq, K_ext, V_ext, Wo)


# device time: 58057 ns/iter; 1.5143x vs baseline; 1.0297x over previous
import jax
import jax.numpy as jnp
from jax import lax
from jax.experimental import pallas as pl
from jax.experimental.pallas import tpu as pltpu

N_DEV = 4
SQ = 256
SKV = 4096
HQ_LOCAL = 8
DH = 128
D_MODEL = 1024
QBLK = 64
NGROUP = 4
KV_PER_GROUP = SKV // NGROUP
NKBLK = KV_PER_GROUP // QBLK
SCALE = 0.08838834764831843


def kernel(x, Wq, K_ext, V_ext, Wo):
    def body(x_ref, wq_ref, k_ref, v_ref, wo_ref, out_ref,
             wq_v, wo_v, kg, vg, q_s, ctx_s,
             sbuf1, sbuf2, sbuf3, rbuf1, rbuf2, rbuf3, rbuf4,
             sem_w, sem_o, sem_kv, send_sems, recv_sems):
        my = lax.axis_index("i")

        wq_dma = pltpu.make_async_copy(
            wq_ref.at[:, pl.ds(my * D_MODEL, D_MODEL)], wq_v, sem_w)
        wq_dma.start()
        wo_dma = pltpu.make_async_copy(
            wo_ref.at[pl.ds(my * D_MODEL, D_MODEL), :], wo_v, sem_o)

        pa = my ^ 1
        pb = 3 - my
        ho = jnp.where((my == 0) | (my == 3), 0, 1)
        qo = jnp.where(my < 2, 0, 1)
        keep_h = ho * 128
        send_h = (1 - ho) * 128
        keep_q = keep_h + qo * 64
        send_q = keep_h + (1 - qo) * 64

        g0 = 2 * (1 - ho)
        g1 = 2 * ho
        order = [g0, g0 + 1, g1, g1 + 1]

        kv_dmas = [[] for _ in range(NGROUP)]
        for s in range(NGROUP):
            g = order[s]
            for t in range(NKBLK):
                boff = (g + NGROUP * t) * QBLK
                kv_dmas[s].append(pltpu.make_async_copy(
                    k_ref.at[0, pl.ds(boff, QBLK)],
                    kg.at[s, pl.ds(t * QBLK, QBLK)], sem_kv.at[s]))
                kv_dmas[s].append(pltpu.make_async_copy(
                    v_ref.at[0, pl.ds(boff, QBLK)],
                    vg.at[s, pl.ds(t * QBLK, QBLK)], sem_kv.at[s]))
        for dg in kv_dmas:
            for d in dg:
                d.start()
        wo_dma.start()

        wq_dma.wait()
        q_s[...] = jnp.dot(
            x_ref[0], wq_v[...], preferred_element_type=jnp.float32) * SCALE

        def attn_slot(s):
            g = order[s]
            for d in kv_dmas[s]:
                d.wait()
            for h in range(HQ_LOCAL):
                qgh = q_s[pl.ds(g * QBLK, QBLK), h * DH:(h + 1) * DH]
                kgh = kg[s, :, h, :]
                sc = lax.dot_general(
                    qgh, kgh, (((1,), (1,)), ((), ())),
                    preferred_element_type=jnp.float32)
                m = jnp.max(sc, axis=1, keepdims=True)
                e = jnp.exp(sc - m)
                w = e / jnp.sum(e, axis=1, keepdims=True)
                vgh = vg[s, :, h, :]
                ctx_s[pl.ds(g * QBLK, QBLK), h * DH:(h + 1) * DH] = (
                    lax.dot_general(
                        w, vgh, (((1,), (0,)), ((), ())),
                        preferred_element_type=jnp.float32))

        attn_slot(0)
        attn_slot(1)
        wo_dma.wait()
        out_ref[0, pl.ds(send_h, 128)] = jnp.dot(
            ctx_s[pl.ds(send_h, 128)], wo_v[...],
            preferred_element_type=jnp.float32)

        sbuf1[...] = out_ref[0, pl.ds(send_h, 128)].astype(jnp.bfloat16)
        r1 = pltpu.make_async_remote_copy(
            src_ref=sbuf1,
            dst_ref=rbuf1,
            send_sem=send_sems.at[0], recv_sem=recv_sems.at[0],
            device_id=(pa,), device_id_type=pl.DeviceIdType.MESH)
        r1.start()

        attn_slot(2)
        attn_slot(3)
        out_ref[0, pl.ds(keep_h, 128)] = jnp.dot(
            ctx_s[pl.ds(keep_h, 128)], wo_v[...],
            preferred_element_type=jnp.float32)

        r1.wait()
        out_ref[0, pl.ds(send_q, 64)] = (
            out_ref[0, pl.ds(send_q, 64)]
            + rbuf1[pl.ds((1 - qo) * 64, 64)].astype(jnp.float32))
        sbuf2[...] = out_ref[0, pl.ds(send_q, 64)].astype(jnp.bfloat16)
        r2 = pltpu.make_async_remote_copy(
            src_ref=sbuf2,
            dst_ref=rbuf2,
            send_sem=send_sems.at[1], recv_sem=recv_sems.at[1],
            device_id=(pb,), device_id_type=pl.DeviceIdType.MESH)
        r2.start()
        out_ref[0, pl.ds(keep_q, 64)] = (
            out_ref[0, pl.ds(keep_q, 64)]
            + rbuf1[pl.ds(qo * 64, 64)].astype(jnp.float32))
        r2.wait()
        out_ref[0, pl.ds(keep_q, 64)] = (
            out_ref[0, pl.ds(keep_q, 64)] + rbuf2[...].astype(jnp.float32))

        sbuf3[...] = out_ref[0, pl.ds(keep_q, 64)].astype(jnp.bfloat16)
        r3 = pltpu.make_async_remote_copy(
            src_ref=sbuf3,
            dst_ref=rbuf3,
            send_sem=send_sems.at[2], recv_sem=recv_sems.at[2],
            device_id=(pb,), device_id_type=pl.DeviceIdType.MESH)
        r3.start()
        r4a = pltpu.make_async_remote_copy(
            src_ref=sbuf3,
            dst_ref=rbuf4.at[pl.ds(qo * 64, 64)],
            send_sem=send_sems.at[3], recv_sem=recv_sems.at[3],
            device_id=(pa,), device_id_type=pl.DeviceIdType.MESH)
        r4a.start()
        r3.wait()
        out_ref[0, pl.ds(send_q, 64)] = rbuf3[...].astype(jnp.float32)
        r4b = pltpu.make_async_remote_copy(
            src_ref=rbuf3,
            dst_ref=rbuf4.at[pl.ds((1 - qo) * 64, 64)],
            send_sem=send_sems.at[4], recv_sem=recv_sems.at[4],
            device_id=(pa,), device_id_type=pl.DeviceIdType.MESH)
        r4b.start()
        r4a.wait()
        r4b.wait()
        out_ref[0, pl.ds(send_h, 128)] = rbuf4[...].astype(jnp.float32)

    out_shape = jax.ShapeDtypeStruct((1, SQ, D_MODEL), jnp.float32)
    return pl.pallas_call(
        body,
        out_shape=out_shape,
        in_specs=[
            pl.BlockSpec(memory_space=pltpu.VMEM),
            pl.BlockSpec(memory_space=pl.ANY),
            pl.BlockSpec(memory_space=pl.ANY),
            pl.BlockSpec(memory_space=pl.ANY),
            pl.BlockSpec(memory_space=pl.ANY),
        ],
        out_specs=pl.BlockSpec(memory_space=pltpu.VMEM),
        scratch_shapes=[
            pltpu.VMEM((D_MODEL, D_MODEL), jnp.float32),
            pltpu.VMEM((D_MODEL, D_MODEL), jnp.float32),
            pltpu.VMEM((NGROUP, KV_PER_GROUP, HQ_LOCAL, DH), jnp.float32),
            pltpu.VMEM((NGROUP, KV_PER_GROUP, HQ_LOCAL, DH), jnp.float32),
            pltpu.VMEM((SQ, D_MODEL), jnp.float32),
            pltpu.VMEM((SQ, D_MODEL), jnp.float32),
            pltpu.VMEM((SQ // 2, D_MODEL), jnp.bfloat16),
            pltpu.VMEM((SQ // 4, D_MODEL), jnp.bfloat16),
            pltpu.VMEM((SQ // 4, D_MODEL), jnp.bfloat16),
            pltpu.VMEM((SQ // 2, D_MODEL), jnp.bfloat16),
            pltpu.VMEM((SQ // 4, D_MODEL), jnp.bfloat16),
            pltpu.VMEM((SQ // 4, D_MODEL), jnp.bfloat16),
            pltpu.VMEM((SQ // 2, D_MODEL), jnp.bfloat16),
            pltpu.SemaphoreType.DMA,
            pltpu.SemaphoreType.DMA,
            pltpu.SemaphoreType.DMA((NGROUP,)),
            pltpu.SemaphoreType.DMA((5,)),
            pltpu.SemaphoreType.DMA((5,)),
        ],
        compiler_params=pltpu.CompilerParams(
            vmem_limit_bytes=100 * 1024 * 1024,
        ),
    )(x, Wq, K_ext, V_ext, Wo)


# device time: 55908 ns/iter; 1.5725x vs baseline; 1.0384x over previous
import jax
import jax.numpy as jnp
from jax import lax
from jax.experimental import pallas as pl
from jax.experimental.pallas import tpu as pltpu

N_DEV = 4
SQ = 256
SKV = 4096
HQ_LOCAL = 8
DH = 128
D_MODEL = 1024
QBLK = 64
NGROUP = 4
KV_PER_GROUP = SKV // NGROUP
NKBLK = KV_PER_GROUP // QBLK
SCALE = 0.08838834764831843


def kernel(x, Wq, K_ext, V_ext, Wo):
    def body(x_ref, wq_ref, k_ref, v_ref, wo_ref, out_ref,
             wq_v, wo_v, kg, vg, q_s, ctx_s,
             sbuf1, sbuf2, sbuf3, rbuf1, rbuf2, rbuf3, rbuf4,
             sem_w, sem_o, sem_kv, send_sems, recv_sems):
        my = lax.axis_index("i")

        wq_dma = pltpu.make_async_copy(
            wq_ref.at[:, pl.ds(my * D_MODEL, D_MODEL)], wq_v, sem_w)
        wq_dma.start()
        wo_dma = pltpu.make_async_copy(
            wo_ref.at[pl.ds(my * D_MODEL, D_MODEL), :], wo_v, sem_o)

        pa = my ^ 1
        pb = 3 - my
        ho = jnp.where((my == 0) | (my == 3), 0, 1)
        qo = jnp.where(my < 2, 0, 1)
        keep_h = ho * 128
        send_h = (1 - ho) * 128
        keep_q = keep_h + qo * 64
        send_q = keep_h + (1 - qo) * 64

        g0 = 2 * (1 - ho)
        g1 = 2 * ho
        order = [g0, g0 + 1, g1 + (1 - qo), g1 + qo]

        kv_dmas = [[] for _ in range(NGROUP)]
        for s in range(NGROUP):
            g = order[s]
            for t in range(NKBLK):
                boff = (g + NGROUP * t) * QBLK
                kv_dmas[s].append(pltpu.make_async_copy(
                    k_ref.at[0, pl.ds(boff, QBLK)],
                    kg.at[s, pl.ds(t * QBLK, QBLK)], sem_kv.at[s]))
                kv_dmas[s].append(pltpu.make_async_copy(
                    v_ref.at[0, pl.ds(boff, QBLK)],
                    vg.at[s, pl.ds(t * QBLK, QBLK)], sem_kv.at[s]))
        for dg in kv_dmas:
            for d in dg:
                d.start()
        wo_dma.start()

        wq_dma.wait()
        q_s[...] = jnp.dot(
            x_ref[0], wq_v[...], preferred_element_type=jnp.float32) * SCALE

        def attn_slot(s):
            g = order[s]
            for d in kv_dmas[s]:
                d.wait()
            for h in range(HQ_LOCAL):
                qgh = q_s[pl.ds(g * QBLK, QBLK), h * DH:(h + 1) * DH]
                kgh = kg[s, :, h, :]
                sc = lax.dot_general(
                    qgh, kgh, (((1,), (1,)), ((), ())),
                    preferred_element_type=jnp.float32)
                m = jnp.max(sc, axis=1, keepdims=True)
                e = jnp.exp(sc - m)
                w = e / jnp.sum(e, axis=1, keepdims=True)
                vgh = vg[s, :, h, :]
                ctx_s[pl.ds(g * QBLK, QBLK), h * DH:(h + 1) * DH] = (
                    lax.dot_general(
                        w, vgh, (((1,), (0,)), ((), ())),
                        preferred_element_type=jnp.float32))

        attn_slot(0)
        attn_slot(1)
        wo_dma.wait()
        out_ref[0, pl.ds(send_h, 128)] = jnp.dot(
            ctx_s[pl.ds(send_h, 128)], wo_v[...],
            preferred_element_type=jnp.float32)

        sbuf1[...] = out_ref[0, pl.ds(send_h, 128)].astype(jnp.bfloat16)
        r1 = pltpu.make_async_remote_copy(
            src_ref=sbuf1,
            dst_ref=rbuf1,
            send_sem=send_sems.at[0], recv_sem=recv_sems.at[0],
            device_id=(pa,), device_id_type=pl.DeviceIdType.MESH)
        r1.start()

        attn_slot(2)
        out_ref[0, pl.ds(send_q, 64)] = jnp.dot(
            ctx_s[pl.ds(send_q, 64)], wo_v[...],
            preferred_element_type=jnp.float32)
        r1.wait()
        out_ref[0, pl.ds(send_q, 64)] = (
            out_ref[0, pl.ds(send_q, 64)]
            + rbuf1[pl.ds((1 - qo) * 64, 64)].astype(jnp.float32))
        sbuf2[...] = out_ref[0, pl.ds(send_q, 64)].astype(jnp.bfloat16)
        r2 = pltpu.make_async_remote_copy(
            src_ref=sbuf2,
            dst_ref=rbuf2,
            send_sem=send_sems.at[1], recv_sem=recv_sems.at[1],
            device_id=(pb,), device_id_type=pl.DeviceIdType.MESH)
        r2.start()
        attn_slot(3)
        out_ref[0, pl.ds(keep_q, 64)] = jnp.dot(
            ctx_s[pl.ds(keep_q, 64)], wo_v[...],
            preferred_element_type=jnp.float32)
        out_ref[0, pl.ds(keep_q, 64)] = (
            out_ref[0, pl.ds(keep_q, 64)]
            + rbuf1[pl.ds(qo * 64, 64)].astype(jnp.float32))
        r2.wait()
        out_ref[0, pl.ds(keep_q, 64)] = (
            out_ref[0, pl.ds(keep_q, 64)] + rbuf2[...].astype(jnp.float32))

        sbuf3[...] = out_ref[0, pl.ds(keep_q, 64)].astype(jnp.bfloat16)
        r3 = pltpu.make_async_remote_copy(
            src_ref=sbuf3,
            dst_ref=rbuf3,
            send_sem=send_sems.at[2], recv_sem=recv_sems.at[2],
            device_id=(pb,), device_id_type=pl.DeviceIdType.MESH)
        r3.start()
        r4a = pltpu.make_async_remote_copy(
            src_ref=sbuf3,
            dst_ref=rbuf4.at[pl.ds(qo * 64, 64)],
            send_sem=send_sems.at[3], recv_sem=recv_sems.at[3],
            device_id=(pa,), device_id_type=pl.DeviceIdType.MESH)
        r4a.start()
        r3.wait()
        out_ref[0, pl.ds(send_q, 64)] = rbuf3[...].astype(jnp.float32)
        r4b = pltpu.make_async_remote_copy(
            src_ref=rbuf3,
            dst_ref=rbuf4.at[pl.ds((1 - qo) * 64, 64)],
            send_sem=send_sems.at[4], recv_sem=recv_sems.at[4],
            device_id=(pa,), device_id_type=pl.DeviceIdType.MESH)
        r4b.start()
        r4a.wait()
        r4b.wait()
        out_ref[0, pl.ds(send_h, 128)] = rbuf4[...].astype(jnp.float32)

    out_shape = jax.ShapeDtypeStruct((1, SQ, D_MODEL), jnp.float32)
    return pl.pallas_call(
        body,
        out_shape=out_shape,
        in_specs=[
            pl.BlockSpec(memory_space=pltpu.VMEM),
            pl.BlockSpec(memory_space=pl.ANY),
            pl.BlockSpec(memory_space=pl.ANY),
            pl.BlockSpec(memory_space=pl.ANY),
            pl.BlockSpec(memory_space=pl.ANY),
        ],
        out_specs=pl.BlockSpec(memory_space=pltpu.VMEM),
        scratch_shapes=[
            pltpu.VMEM((D_MODEL, D_MODEL), jnp.float32),
            pltpu.VMEM((D_MODEL, D_MODEL), jnp.float32),
            pltpu.VMEM((NGROUP, KV_PER_GROUP, HQ_LOCAL, DH), jnp.float32),
            pltpu.VMEM((NGROUP, KV_PER_GROUP, HQ_LOCAL, DH), jnp.float32),
            pltpu.VMEM((SQ, D_MODEL), jnp.float32),
            pltpu.VMEM((SQ, D_MODEL), jnp.float32),
            pltpu.VMEM((SQ // 2, D_MODEL), jnp.bfloat16),
            pltpu.VMEM((SQ // 4, D_MODEL), jnp.bfloat16),
            pltpu.VMEM((SQ // 4, D_MODEL), jnp.bfloat16),
            pltpu.VMEM((SQ // 2, D_MODEL), jnp.bfloat16),
            pltpu.VMEM((SQ // 4, D_MODEL), jnp.bfloat16),
            pltpu.VMEM((SQ // 4, D_MODEL), jnp.bfloat16),
            pltpu.VMEM((SQ // 2, D_MODEL), jnp.bfloat16),
            pltpu.SemaphoreType.DMA,
            pltpu.SemaphoreType.DMA,
            pltpu.SemaphoreType.DMA((NGROUP,)),
            pltpu.SemaphoreType.DMA((5,)),
            pltpu.SemaphoreType.DMA((5,)),
        ],
        compiler_params=pltpu.CompilerParams(
            vmem_limit_bytes=100 * 1024 * 1024,
        ),
    )(x, Wq, K_ext, V_ext, Wo)
